# Initial kernel scaffold; baseline (speedup 1.0000x reference)
import jax
import jax.numpy as jnp
from jax.experimental import pallas as pl


def kernel(xyzs, pointcloud_features, level):
    raise NotImplementedError("write your pallas kernel here")


# speedup vs baseline: 1.0134x; 1.0134x over previous
import functools

import jax
import jax.numpy as jnp
from jax.experimental import pallas as pl

_B = 2
_N = 40000
_DIM = 768
_G = 256
_M = 64


def _fps(xyz, n_samples):
    Bb, Nn, _ = xyz.shape
    centroids = jnp.zeros((Bb, n_samples), dtype=jnp.int32)
    distance = jnp.full((Bb, Nn), 1e10, dtype=xyz.dtype)
    farthest = jnp.zeros((Bb,), dtype=jnp.int32)

    def body(i, carry):
        centroids, distance, farthest = carry
        centroids = centroids.at[:, i].set(farthest)
        centroid = xyz[jnp.arange(Bb), farthest][:, None, :]
        dist = jnp.sum((xyz - centroid) ** 2, axis=-1)
        distance = jnp.minimum(distance, dist)
        farthest = jnp.argmax(distance, axis=-1).astype(jnp.int32)
        return centroids, distance, farthest

    centroids, _, _ = jax.lax.fori_loop(0, n_samples, body, (centroids, distance, farthest))
    return centroids


def _mean_body(x_ref, o_ref):
    o_ref[...] = jnp.mean(x_ref[...], axis=1)


def _pallas_mean(nb_fts):
    return pl.pallas_call(
        _mean_body,
        grid=(nb_fts.shape[0] // 8,),
        in_specs=[pl.BlockSpec((8, _M, _DIM), lambda i: (i, 0, 0))],
        out_specs=pl.BlockSpec((8, _DIM), lambda i: (i, 0)),
        out_shape=jax.ShapeDtypeStruct((nb_fts.shape[0], _DIM), nb_fts.dtype),
    )(nb_fts)


def kernel(xyzs, pointcloud_features, level):
    Bb, _, dim = pointcloud_features.shape
    xyz = xyzs[:, :_N, :]
    fps_idx = _fps(xyz, _G)
    bidx = jnp.arange(Bb)[:, None]
    center = xyz[bidx, fps_idx]
    d2 = (jnp.sum(center ** 2, axis=-1)[:, :, None]
          - 2.0 * jnp.einsum('bgc,bnc->bgn', center, xyz)
          + jnp.sum(xyz ** 2, axis=-1)[:, None, :])
    _, idx = jax.lax.top_k(-d2, _M)
    bidx2 = jnp.arange(Bb)[:, None, None]
    nxyz = xyz[bidx2, idx] - center[:, :, None, :]
    nb_fts = pointcloud_features[bidx2, idx]
    scene_fts = _pallas_mean(nb_fts.reshape(Bb * _G, _M, dim)).reshape(Bb, _G, dim)
    all_fts_mask = jnp.ones((Bb, _G), dtype=pointcloud_features.dtype)
    return scene_fts, all_fts_mask, center, nxyz


# speedup vs baseline: 3.7553x; 3.7056x over previous
import functools

import jax
import jax.numpy as jnp
from jax import lax
from jax.experimental import pallas as pl
from jax.experimental.pallas import tpu as pltpu
from jax.experimental.pallas import tpu_sc as plsc

_B = 2
_N = 40000
_DIM = 768
_G = 256
_M = 64

_SUB = 8
_LANES = _N // _SUB

_NC = 2
_NS = 16
_NW = _NC * _NS
_GPW = (_B * _G) // _NW


def _fps_body(xr_ref, ctr_ref):
    x = xr_ref[0, 0]
    y = xr_ref[0, 1]
    z = xr_ref[0, 2]
    r_iota = lax.broadcasted_iota(jnp.int32, (_SUB, _LANES), 0)
    c_iota = lax.broadcasted_iota(jnp.int32, (_SUB, _LANES), 1)
    flat = r_iota * _LANES + c_iota

    def body(i, carry):
        dist_run, far = carry
        mask = flat == far
        cx = jnp.sum(jnp.where(mask, x, 0.0))
        cy = jnp.sum(jnp.where(mask, y, 0.0))
        cz = jnp.sum(jnp.where(mask, z, 0.0))
        ctr_ref[0, pl.ds(i, 1), :] = jnp.concatenate(
            [cx.reshape(1, 1), cy.reshape(1, 1), cz.reshape(1, 1)], axis=1)
        d = (x - cx) ** 2 + (y - cy) ** 2 + (z - cz) ** 2
        dist_run = jnp.minimum(dist_run, d)
        m = jnp.max(dist_run)
        far2 = jnp.min(jnp.where(dist_run == m, flat, jnp.int32(2 ** 30)))
        return dist_run, far2

    lax.fori_loop(0, _G, body,
                  (jnp.full((_SUB, _LANES), 1e10, jnp.float32), jnp.int32(0)))


def _fps(xr):
    return pl.pallas_call(
        _fps_body,
        grid=(_B,),
        in_specs=[pl.BlockSpec((1, 3, _SUB, _LANES), lambda b: (b, 0, 0, 0))],
        out_specs=pl.BlockSpec((1, _G, 3), lambda b: (b, 0, 0)),
        out_shape=jax.ShapeDtypeStruct((_B, _G, 3), jnp.float32),
    )(xr)


_GB = _G // _SUB
_NP = 40960


def _knn_body(xf_ref, ctr_ref, idx_ref, d_ref):
    x = xf_ref[0, 0:1, :]
    y = xf_ref[0, 1:2, :]
    z = xf_ref[0, 2:3, :]
    cx = ctr_ref[0, :, 0:1]
    cy = ctr_ref[0, :, 1:2]
    cz = ctr_ref[0, :, 2:3]
    xs = x * x + y * y + z * z
    cs = cx * cx + cy * cy + cz * cz
    xb = x.astype(jnp.bfloat16).astype(jnp.float32)
    yb = y.astype(jnp.bfloat16).astype(jnp.float32)
    zb = z.astype(jnp.bfloat16).astype(jnp.float32)
    cxb = cx.astype(jnp.bfloat16).astype(jnp.float32)
    cyb = cy.astype(jnp.bfloat16).astype(jnp.float32)
    czb = cz.astype(jnp.bfloat16).astype(jnp.float32)
    lane = lax.broadcasted_iota(jnp.int32, (_SUB, _NP), 1)
    d2 = cs - 2.0 * (cxb * xb + cyb * yb + czb * zb) + xs
    d_ref[...] = jnp.where(lane < _N, d2, jnp.inf)
    col = lax.broadcasted_iota(jnp.int32, (_SUB, _M), 1)

    def body(j, acc):
        d = d_ref[...]
        m = jnp.min(d, axis=1, keepdims=True)
        sel = jnp.where(d == m, lane, jnp.int32(2 ** 30))
        idx8 = jnp.min(sel, axis=1, keepdims=True)
        d_ref[...] = jnp.where(lane == idx8, jnp.inf, d)
        return jnp.where(col == j, idx8, acc)

    acc = lax.fori_loop(0, _M, body,
                        jnp.zeros((_SUB, _M), jnp.int32))
    idx_ref[0, 0] = acc


def _knn(xf, center):
    return pl.pallas_call(
        _knn_body,
        grid=(_B, _GB),
        in_specs=[
            pl.BlockSpec((1, 3, _NP), lambda b, g: (b, 0, 0)),
            pl.BlockSpec((1, _SUB, 3), lambda b, g: (b, g, 0)),
        ],
        out_specs=pl.BlockSpec((1, 1, _SUB, _M), lambda b, g: (b, g, 0, 0)),
        out_shape=jax.ShapeDtypeStruct((_B, _GB, _SUB, _M), jnp.int32),
        scratch_shapes=[pltpu.VMEM((_SUB, _NP), jnp.float32)],
    )(xf, center)


_DCH = _DIM // 16


def _sc_body(feat_hbm, xyzp_hbm, idx_hbm, ctrp_hbm, fts_out, nxyz_out,
             idx_v, rows_v, xrows_v, ctr_v, acc_v, sem1, sem2):
    wid = lax.axis_index("s") * _NC + lax.axis_index("c")
    base = wid * _GPW

    def group_body(gl, _):
        g = pl.multiple_of(base + gl, 1)
        o64 = pl.multiple_of(g * _M, 8)
        pltpu.sync_copy(idx_hbm.at[pl.ds(o64, _M)], idx_v)
        pltpu.async_copy(feat_hbm.at[idx_v], rows_v, sem1)
        pltpu.async_copy(xyzp_hbm.at[idx_v], xrows_v, sem2)
        pltpu.sync_copy(ctrp_hbm.at[g], ctr_v)
        pltpu.make_async_copy(feat_hbm.at[idx_v], rows_v, sem1).wait()

        def red_body(mi, accs):
            return tuple(
                accs[dc] + rows_v[mi, pl.ds(dc * 16, 16)]
                for dc in range(_DCH))

        accs = lax.fori_loop(
            0, _M, red_body,
            tuple(jnp.zeros((16,), jnp.float32) for _ in range(_DCH)))
        for dc in range(_DCH):
            acc_v[pl.ds(dc * 16, 16)] = accs[dc] * (1.0 / _M)
        pltpu.sync_copy(acc_v, fts_out.at[g])

        pltpu.make_async_copy(xyzp_hbm.at[idx_v], xrows_v, sem2).wait()
        cvec = ctr_v[...]

        def sub_body(mi, _c):
            xrows_v[mi, :] = xrows_v[mi, :] - cvec
            return 0

        lax.fori_loop(0, _M, sub_body, 0)
        pltpu.sync_copy(xrows_v, nxyz_out.at[pl.ds(o64, _M)])
        return 0

    lax.fori_loop(0, _GPW, group_body, 0)


def _sc_gather_mean(feat_tab, xyzp_tab, flat_idx, ctrp):
    mesh = plsc.VectorSubcoreMesh(core_axis_name="c", subcore_axis_name="s")
    return pl.kernel(
        _sc_body,
        mesh=mesh,
        out_type=[
            jax.ShapeDtypeStruct((_B * _G, _DIM), jnp.float32),
            jax.ShapeDtypeStruct((_B * _G * _M, 16), jnp.float32),
        ],
        scratch_types=[
            pltpu.VMEM((_M,), jnp.int32),
            pltpu.VMEM((_M, _DIM), jnp.float32),
            pltpu.VMEM((_M, 16), jnp.float32),
            pltpu.VMEM((16,), jnp.float32),
            pltpu.VMEM((_DIM,), jnp.float32),
            pltpu.SemaphoreType.DMA,
            pltpu.SemaphoreType.DMA,
        ],
    )(feat_tab, xyzp_tab, flat_idx, ctrp)


def kernel(xyzs, pointcloud_features, level):
    Bb, Nn, dim = pointcloud_features.shape
    xt = jnp.transpose(xyzs, (0, 2, 1))
    xr = xt.reshape(Bb, 3, _SUB, _LANES)
    center = _fps(xr)
    xtp = jnp.pad(xt, ((0, 0), (0, 0), (0, _NP - _N)),
                  constant_values=1e6)
    idx4 = _knn(xtp, center)
    idx = idx4.reshape(Bb, _G, _M)
    flat_idx = (idx + (jnp.arange(Bb, dtype=jnp.int32) * Nn)[:, None, None])
    flat_idx = flat_idx.reshape(-1).astype(jnp.int32)
    _USE_SC = False
    if _USE_SC:
        feat_tab = pointcloud_features.reshape(Bb * Nn, dim)
        xyzp_tab = jnp.pad(xyzs, ((0, 0), (0, 0), (0, 13))).reshape(Bb * Nn, 16)
        ctrp = jnp.pad(center, ((0, 0), (0, 0), (0, 13))).reshape(Bb * _G, 16)
        fts, nxyzp = _sc_gather_mean(feat_tab, xyzp_tab, flat_idx, ctrp)
        all_fts = fts.reshape(Bb, _G, dim)
        nxyz = nxyzp[:, :3].reshape(Bb, _G, _M, 3)
    else:
        bidx2 = jnp.arange(Bb)[:, None, None]
        all_fts = pointcloud_features[bidx2, idx].mean(-2)
        nxyz = xyzs[bidx2, idx] - center[:, :, None, :]
    all_fts_mask = jnp.ones((Bb, _G), dtype=pointcloud_features.dtype)
    return all_fts, all_fts_mask, center, nxyz


# speedup vs baseline: 3.7670x; 1.0031x over previous
import functools

import jax
import jax.numpy as jnp
from jax import lax
from jax.experimental import pallas as pl
from jax.experimental.pallas import tpu as pltpu
from jax.experimental.pallas import tpu_sc as plsc

_B = 2
_N = 40000
_DIM = 768
_G = 256
_M = 64

_SUB = 8
_LANES = _N // _SUB

_NC = 2
_NS = 16
_NW = _NC * _NS
_GPW = (_B * _G) // _NW


def _fps_body(xr_ref, ctr_ref):
    x = xr_ref[0, 0]
    y = xr_ref[0, 1]
    z = xr_ref[0, 2]
    r_iota = lax.broadcasted_iota(jnp.int32, (_SUB, _LANES), 0)
    c_iota = lax.broadcasted_iota(jnp.int32, (_SUB, _LANES), 1)
    flat = r_iota * _LANES + c_iota

    def body(i, carry):
        dist_run, far = carry
        mask = flat == far
        cx = jnp.sum(jnp.where(mask, x, 0.0))
        cy = jnp.sum(jnp.where(mask, y, 0.0))
        cz = jnp.sum(jnp.where(mask, z, 0.0))
        ctr_ref[0, pl.ds(i, 1), :] = jnp.concatenate(
            [cx.reshape(1, 1), cy.reshape(1, 1), cz.reshape(1, 1)], axis=1)
        d = (x - cx) ** 2 + (y - cy) ** 2 + (z - cz) ** 2
        dist_run = jnp.minimum(dist_run, d)
        m = jnp.max(dist_run)
        far2 = jnp.min(jnp.where(dist_run == m, flat, jnp.int32(2 ** 30)))
        return dist_run, far2

    lax.fori_loop(0, _G, body,
                  (jnp.full((_SUB, _LANES), 1e10, jnp.float32), jnp.int32(0)))


def _fps(xr):
    return pl.pallas_call(
        _fps_body,
        grid=(_B,),
        in_specs=[pl.BlockSpec((1, 3, _SUB, _LANES), lambda b: (b, 0, 0, 0))],
        out_specs=pl.BlockSpec((1, _G, 3), lambda b: (b, 0, 0)),
        out_shape=jax.ShapeDtypeStruct((_B, _G, 3), jnp.float32),
    )(xr)


_GB = _G // _SUB
_NP = 40960


def _knn_body(xf_ref, ctr_ref, idx_ref, d_ref):
    x = xf_ref[0, 0:1, :]
    y = xf_ref[0, 1:2, :]
    z = xf_ref[0, 2:3, :]
    cx = ctr_ref[0, :, 0:1]
    cy = ctr_ref[0, :, 1:2]
    cz = ctr_ref[0, :, 2:3]
    xs = x * x + y * y + z * z
    cs = cx * cx + cy * cy + cz * cz
    xb = x.astype(jnp.bfloat16).astype(jnp.float32)
    yb = y.astype(jnp.bfloat16).astype(jnp.float32)
    zb = z.astype(jnp.bfloat16).astype(jnp.float32)
    cxb = cx.astype(jnp.bfloat16).astype(jnp.float32)
    cyb = cy.astype(jnp.bfloat16).astype(jnp.float32)
    czb = cz.astype(jnp.bfloat16).astype(jnp.float32)
    lane = lax.broadcasted_iota(jnp.int32, (_SUB, _NP), 1)
    d2 = cs - 2.0 * (cxb * xb + cyb * yb + czb * zb) + xs
    d_ref[...] = jnp.where(lane < _N, d2, jnp.inf)
    col = lax.broadcasted_iota(jnp.int32, (_SUB, _M), 1)

    def body(j, acc):
        d = d_ref[...]
        m = jnp.min(d, axis=1, keepdims=True)
        sel = jnp.where(d == m, lane, jnp.int32(2 ** 30))
        idx8 = jnp.min(sel, axis=1, keepdims=True)
        d_ref[...] = jnp.where(lane == idx8, jnp.inf, d)
        return jnp.where(col == j, idx8, acc)

    acc = lax.fori_loop(0, _M, body,
                        jnp.zeros((_SUB, _M), jnp.int32))
    idx_ref[0, 0] = acc


def _knn(xf, center):
    return pl.pallas_call(
        _knn_body,
        grid=(_B, _GB),
        in_specs=[
            pl.BlockSpec((1, 3, _NP), lambda b, g: (b, 0, 0)),
            pl.BlockSpec((1, _SUB, 3), lambda b, g: (b, g, 0)),
        ],
        out_specs=pl.BlockSpec((1, 1, _SUB, _M), lambda b, g: (b, g, 0, 0)),
        out_shape=jax.ShapeDtypeStruct((_B, _GB, _SUB, _M), jnp.int32),
        scratch_shapes=[pltpu.VMEM((_SUB, _NP), jnp.float32)],
    )(xf, center)


_DCH = _DIM // 16


def _sc_body(feat_hbm, xyzp_hbm, idx_hbm, ctrp_hbm, fts_out, nxyz_out,
             idx_v, rows_v, xrows_v, ctr_v, acc_v, sem1, sem2):
    wid = lax.axis_index("s") * _NC + lax.axis_index("c")
    base = wid * _GPW

    def group_body(gl, _):
        g = pl.multiple_of(base + gl, 1)
        o64 = pl.multiple_of(g * _M, 8)
        pltpu.sync_copy(idx_hbm.at[pl.ds(o64, _M)], idx_v)
        pltpu.async_copy(feat_hbm.at[idx_v], rows_v, sem1)
        pltpu.async_copy(xyzp_hbm.at[idx_v], xrows_v, sem2)
        pltpu.sync_copy(ctrp_hbm.at[g], ctr_v)
        pltpu.make_async_copy(feat_hbm.at[idx_v], rows_v, sem1).wait()

        def red_body(mi, accs):
            return tuple(
                accs[dc] + rows_v[mi, pl.ds(dc * 16, 16)]
                for dc in range(_DCH))

        accs = lax.fori_loop(
            0, _M, red_body,
            tuple(jnp.zeros((16,), jnp.float32) for _ in range(_DCH)))
        for dc in range(_DCH):
            acc_v[pl.ds(dc * 16, 16)] = accs[dc] * (1.0 / _M)
        pltpu.sync_copy(acc_v, fts_out.at[g])

        pltpu.make_async_copy(xyzp_hbm.at[idx_v], xrows_v, sem2).wait()
        cvec = ctr_v[...]

        def sub_body(mi, _c):
            xrows_v[mi, 0:16] = xrows_v[mi, 0:16] - cvec
            return 0

        lax.fori_loop(0, _M, sub_body, 0)
        pltpu.sync_copy(xrows_v, nxyz_out.at[pl.ds(o64, _M)])
        return 0

    lax.fori_loop(0, _GPW, group_body, 0)


def _sc_gather_mean(feat_tab, xyzp_tab, flat_idx, ctrp):
    mesh = plsc.VectorSubcoreMesh(core_axis_name="c", subcore_axis_name="s")
    return pl.kernel(
        _sc_body,
        mesh=mesh,
        out_type=[
            jax.ShapeDtypeStruct((_B * _G, _DIM), jnp.float32),
            jax.ShapeDtypeStruct((_B * _G * _M, 128), jnp.float32),
        ],
        scratch_types=[
            pltpu.VMEM((_M,), jnp.int32),
            pltpu.VMEM((_M, _DIM), jnp.float32),
            pltpu.VMEM((_M, 128), jnp.float32),
            pltpu.VMEM((16,), jnp.float32),
            pltpu.VMEM((_DIM,), jnp.float32),
            pltpu.SemaphoreType.DMA,
            pltpu.SemaphoreType.DMA,
        ],
    )(feat_tab, xyzp_tab, flat_idx, ctrp)


def kernel(xyzs, pointcloud_features, level):
    Bb, Nn, dim = pointcloud_features.shape
    xt = jnp.transpose(xyzs, (0, 2, 1))
    xr = xt.reshape(Bb, 3, _SUB, _LANES)
    center = _fps(xr)
    xtp = jnp.pad(xt, ((0, 0), (0, 0), (0, _NP - _N)),
                  constant_values=1e6)
    idx4 = _knn(xtp, center)
    idx = idx4.reshape(Bb, _G, _M)
    flat_idx = (idx + (jnp.arange(Bb, dtype=jnp.int32) * Nn)[:, None, None])
    flat_idx = flat_idx.reshape(-1).astype(jnp.int32)
    _USE_SC = True
    if _USE_SC:
        feat_tab = pointcloud_features.reshape(Bb * Nn, dim)
        xyzp_tab = jnp.pad(xyzs, ((0, 0), (0, 0), (0, 125))).reshape(Bb * Nn, 128)
        ctrp = jnp.pad(center, ((0, 0), (0, 0), (0, 13))).reshape(Bb * _G, 16)
        fts, nxyzp = _sc_gather_mean(feat_tab, xyzp_tab, flat_idx, ctrp)
        all_fts = fts.reshape(Bb, _G, dim)
        nxyz = nxyzp[:, :3].reshape(Bb, _G, _M, 3)
    else:
        bidx2 = jnp.arange(Bb)[:, None, None]
        all_fts = pointcloud_features[bidx2, idx].mean(-2)
        nxyz = xyzs[bidx2, idx] - center[:, :, None, :]
    all_fts_mask = jnp.ones((Bb, _G), dtype=pointcloud_features.dtype)
    return all_fts, all_fts_mask, center, nxyz


# speedup vs baseline: 5.8830x; 1.5617x over previous
import functools

import jax
import jax.numpy as jnp
from jax import lax
from jax.experimental import pallas as pl
from jax.experimental.pallas import tpu as pltpu
from jax.experimental.pallas import tpu_sc as plsc

_B = 2
_N = 40000
_DIM = 768
_G = 256
_M = 64

_SUB = 8
_LANES = _N // _SUB

_NC = 2
_NS = 16
_NW = _NC * _NS
_GPW = (_B * _G) // _NW


def _fps_body(xr_ref, ctr_ref):
    x = xr_ref[0, 0]
    y = xr_ref[0, 1]
    z = xr_ref[0, 2]
    r_iota = lax.broadcasted_iota(jnp.int32, (_SUB, _LANES), 0)
    c_iota = lax.broadcasted_iota(jnp.int32, (_SUB, _LANES), 1)
    flat = r_iota * _LANES + c_iota

    def body(i, carry):
        dist_run, far = carry
        mask = flat == far
        cx = jnp.sum(jnp.where(mask, x, 0.0))
        cy = jnp.sum(jnp.where(mask, y, 0.0))
        cz = jnp.sum(jnp.where(mask, z, 0.0))
        ctr_ref[0, pl.ds(i, 1), :] = jnp.concatenate(
            [cx.reshape(1, 1), cy.reshape(1, 1), cz.reshape(1, 1)], axis=1)
        d = (x - cx) ** 2 + (y - cy) ** 2 + (z - cz) ** 2
        dist_run = jnp.minimum(dist_run, d)
        m = jnp.max(dist_run)
        far2 = jnp.min(jnp.where(dist_run == m, flat, jnp.int32(2 ** 30)))
        return dist_run, far2

    lax.fori_loop(0, _G, body,
                  (jnp.full((_SUB, _LANES), 1e10, jnp.float32), jnp.int32(0)))


def _fps(xr):
    return pl.pallas_call(
        _fps_body,
        grid=(_B,),
        in_specs=[pl.BlockSpec((1, 3, _SUB, _LANES), lambda b: (b, 0, 0, 0))],
        out_specs=pl.BlockSpec((1, _G, 3), lambda b: (b, 0, 0)),
        out_shape=jax.ShapeDtypeStruct((_B, _G, 3), jnp.float32),
    )(xr)


_KG = 16
_GBK = _G // _KG
_NP = 40960


def _knn_body(xf_ref, ctr_ref, idx_ref, d_ref):
    x = xf_ref[0, 0:1, :]
    y = xf_ref[0, 1:2, :]
    z = xf_ref[0, 2:3, :]
    cx = ctr_ref[0, :, 0:1]
    cy = ctr_ref[0, :, 1:2]
    cz = ctr_ref[0, :, 2:3]
    xs = x * x + y * y + z * z
    cs = cx * cx + cy * cy + cz * cz
    xb = x.astype(jnp.bfloat16).astype(jnp.float32)
    yb = y.astype(jnp.bfloat16).astype(jnp.float32)
    zb = z.astype(jnp.bfloat16).astype(jnp.float32)
    cxb = cx.astype(jnp.bfloat16).astype(jnp.float32)
    cyb = cy.astype(jnp.bfloat16).astype(jnp.float32)
    czb = cz.astype(jnp.bfloat16).astype(jnp.float32)
    lane = lax.broadcasted_iota(jnp.int32, (_KG, _NP), 1)
    d2 = cs - 2.0 * (cxb * xb + cyb * yb + czb * zb) + xs
    d_ref[...] = jnp.where(lane < _N, d2, jnp.inf)
    col = lax.broadcasted_iota(jnp.int32, (_KG, _M), 1)

    def body(j, acc):
        d = d_ref[...]
        m = jnp.min(d, axis=1, keepdims=True)
        sel = jnp.where(d == m, lane, jnp.int32(2 ** 30))
        idx8 = jnp.min(sel, axis=1, keepdims=True)
        d_ref[...] = jnp.where(lane == idx8, jnp.inf, d)
        return jnp.where(col == j, idx8, acc)

    acc = lax.fori_loop(0, _M, body,
                        jnp.zeros((_KG, _M), jnp.int32))
    idx_ref[0, 0] = acc


def _knn(xf, center):
    return pl.pallas_call(
        _knn_body,
        grid=(_B, _GBK),
        in_specs=[
            pl.BlockSpec((1, 3, _NP), lambda b, g: (b, 0, 0)),
            pl.BlockSpec((1, _KG, 3), lambda b, g: (b, g, 0)),
        ],
        out_specs=pl.BlockSpec((1, 1, _KG, _M), lambda b, g: (b, g, 0, 0)),
        out_shape=jax.ShapeDtypeStruct((_B, _GBK, _KG, _M), jnp.int32),
        scratch_shapes=[pltpu.VMEM((_KG, _NP), jnp.float32)],
    )(xf, center)


_DCH = _DIM // 16


def _sc_body(feat_hbm, xyzp_hbm, idx_hbm, ctrp_hbm, fts_out, nxyz_out,
             idx_v, rows_v, xrows_v, ctr_v, acc_v, sem1, sem2):
    wid = lax.axis_index("s") * _NC + lax.axis_index("c")
    base = wid * _GPW

    def group_body(gl, _):
        g = pl.multiple_of(base + gl, 1)
        o64 = pl.multiple_of(g * _M, 8)
        pltpu.sync_copy(idx_hbm.at[pl.ds(o64, _M)], idx_v)
        pltpu.async_copy(feat_hbm.at[idx_v], rows_v, sem1)
        pltpu.async_copy(xyzp_hbm.at[idx_v], xrows_v, sem2)
        pltpu.sync_copy(ctrp_hbm.at[g], ctr_v)
        pltpu.make_async_copy(feat_hbm.at[idx_v], rows_v, sem1).wait()

        def red_body(mi, accs):
            return tuple(
                accs[dc] + rows_v[mi, pl.ds(dc * 16, 16)]
                for dc in range(_DCH))

        accs = lax.fori_loop(
            0, _M, red_body,
            tuple(jnp.zeros((16,), jnp.float32) for _ in range(_DCH)))
        for dc in range(_DCH):
            acc_v[pl.ds(dc * 16, 16)] = accs[dc] * (1.0 / _M)
        pltpu.sync_copy(acc_v, fts_out.at[g])

        pltpu.make_async_copy(xyzp_hbm.at[idx_v], xrows_v, sem2).wait()
        cvec = ctr_v[...]

        def sub_body(mi, _c):
            xrows_v[mi, 0:16] = xrows_v[mi, 0:16] - cvec
            return 0

        lax.fori_loop(0, _M, sub_body, 0)
        pltpu.sync_copy(xrows_v, nxyz_out.at[pl.ds(o64, _M)])
        return 0

    lax.fori_loop(0, _GPW, group_body, 0)


def _sc_gather_mean(feat_tab, xyzp_tab, flat_idx, ctrp):
    mesh = plsc.VectorSubcoreMesh(core_axis_name="c", subcore_axis_name="s")
    return pl.kernel(
        _sc_body,
        mesh=mesh,
        out_type=[
            jax.ShapeDtypeStruct((_B * _G, _DIM), jnp.float32),
            jax.ShapeDtypeStruct((_B * _G * _M, 128), jnp.float32),
        ],
        scratch_types=[
            pltpu.VMEM((_M,), jnp.int32),
            pltpu.VMEM((_M, _DIM), jnp.float32),
            pltpu.VMEM((_M, 128), jnp.float32),
            pltpu.VMEM((16,), jnp.float32),
            pltpu.VMEM((_DIM,), jnp.float32),
            pltpu.SemaphoreType.DMA,
            pltpu.SemaphoreType.DMA,
        ],
    )(feat_tab, xyzp_tab, flat_idx, ctrp)


def kernel(xyzs, pointcloud_features, level):
    Bb, Nn, dim = pointcloud_features.shape
    xt = jnp.transpose(xyzs, (0, 2, 1))
    xr = xt.reshape(Bb, 3, _SUB, _LANES)
    center = _fps(xr)
    xtp = jnp.pad(xt, ((0, 0), (0, 0), (0, _NP - _N)),
                  constant_values=1e6)
    idx4 = _knn(xtp, center)
    idx = idx4.reshape(Bb, _G, _M)
    flat_idx = (idx + (jnp.arange(Bb, dtype=jnp.int32) * Nn)[:, None, None])
    flat_idx = flat_idx.reshape(-1).astype(jnp.int32)
    _USE_SC = True
    if _USE_SC:
        feat_tab = pointcloud_features.reshape(Bb * Nn, dim)
        xyzp_tab = jnp.pad(xyzs, ((0, 0), (0, 0), (0, 125))).reshape(Bb * Nn, 128)
        ctrp = jnp.pad(center, ((0, 0), (0, 0), (0, 13))).reshape(Bb * _G, 16)
        fts, nxyzp = _sc_gather_mean(feat_tab, xyzp_tab, flat_idx, ctrp)
        all_fts = fts.reshape(Bb, _G, dim)
        nxyz = nxyzp[:, :3].reshape(Bb, _G, _M, 3)
    else:
        bidx2 = jnp.arange(Bb)[:, None, None]
        all_fts = pointcloud_features[bidx2, idx].mean(-2)
        nxyz = xyzs[bidx2, idx] - center[:, :, None, :]
    all_fts_mask = jnp.ones((Bb, _G), dtype=pointcloud_features.dtype)
    return all_fts, all_fts_mask, center, nxyz


# speedup vs baseline: 6.5375x; 1.1112x over previous
import functools

import jax
import jax.numpy as jnp
from jax import lax
from jax.experimental import pallas as pl
from jax.experimental.pallas import tpu as pltpu
from jax.experimental.pallas import tpu_sc as plsc

_B = 2
_N = 40000
_DIM = 768
_G = 256
_M = 64

_SUB = 8
_LANES = _N // _SUB

_NC = 2
_NS = 16
_NW = _NC * _NS
_GPW = (_B * _G) // _NW


def _fps_body(xr_ref, ctr_ref):
    x = xr_ref[0, 0]
    y = xr_ref[0, 1]
    z = xr_ref[0, 2]
    r_iota = lax.broadcasted_iota(jnp.int32, (_SUB, _LANES), 0)
    c_iota = lax.broadcasted_iota(jnp.int32, (_SUB, _LANES), 1)
    flat = r_iota * _LANES + c_iota

    def body(i, carry):
        dist_run, far = carry
        mask = flat == far
        cx = jnp.sum(jnp.where(mask, x, 0.0))
        cy = jnp.sum(jnp.where(mask, y, 0.0))
        cz = jnp.sum(jnp.where(mask, z, 0.0))
        ctr_ref[0, pl.ds(i, 1), :] = jnp.concatenate(
            [cx.reshape(1, 1), cy.reshape(1, 1), cz.reshape(1, 1)], axis=1)
        d = (x - cx) ** 2 + (y - cy) ** 2 + (z - cz) ** 2
        dist_run = jnp.minimum(dist_run, d)
        m = jnp.max(dist_run)
        far2 = jnp.min(jnp.where(dist_run == m, flat, jnp.int32(2 ** 30)))
        return dist_run, far2

    lax.fori_loop(0, _G, body,
                  (jnp.full((_SUB, _LANES), 1e10, jnp.float32), jnp.int32(0)))


def _fps(xr):
    return pl.pallas_call(
        _fps_body,
        grid=(_B,),
        in_specs=[pl.BlockSpec((1, 3, _SUB, _LANES), lambda b: (b, 0, 0, 0))],
        out_specs=pl.BlockSpec((1, _G, 3), lambda b: (b, 0, 0)),
        out_shape=jax.ShapeDtypeStruct((_B, _G, 3), jnp.float32),
    )(xr)


_KG = 32
_GBK = _G // _KG
_NP = 40960


def _knn_body(xf_ref, ctr_ref, idx_ref, d_ref):
    x = xf_ref[0, 0:1, :]
    y = xf_ref[0, 1:2, :]
    z = xf_ref[0, 2:3, :]
    cx = ctr_ref[0, :, 0:1]
    cy = ctr_ref[0, :, 1:2]
    cz = ctr_ref[0, :, 2:3]
    xs = x * x + y * y + z * z
    cs = cx * cx + cy * cy + cz * cz
    xb = x.astype(jnp.bfloat16).astype(jnp.float32)
    yb = y.astype(jnp.bfloat16).astype(jnp.float32)
    zb = z.astype(jnp.bfloat16).astype(jnp.float32)
    cxb = cx.astype(jnp.bfloat16).astype(jnp.float32)
    cyb = cy.astype(jnp.bfloat16).astype(jnp.float32)
    czb = cz.astype(jnp.bfloat16).astype(jnp.float32)
    lane = lax.broadcasted_iota(jnp.int32, (_KG, _NP), 1)
    d2 = cs - 2.0 * (cxb * xb + cyb * yb + czb * zb) + xs
    d_ref[...] = jnp.where(lane < _N, d2, jnp.inf)
    col = lax.broadcasted_iota(jnp.int32, (_KG, _M), 1)

    def body(j, acc):
        d = d_ref[...]
        m = jnp.min(d, axis=1, keepdims=True)
        sel = jnp.where(d == m, lane, jnp.int32(2 ** 30))
        idx8 = jnp.min(sel, axis=1, keepdims=True)
        d_ref[...] = jnp.where(lane == idx8, jnp.inf, d)
        return jnp.where(col == j, idx8, acc)

    acc = lax.fori_loop(0, _M, body,
                        jnp.zeros((_KG, _M), jnp.int32))
    idx_ref[0, 0] = acc


def _knn(xf, center):
    return pl.pallas_call(
        _knn_body,
        grid=(_B, _GBK),
        in_specs=[
            pl.BlockSpec((1, 3, _NP), lambda b, g: (b, 0, 0)),
            pl.BlockSpec((1, _KG, 3), lambda b, g: (b, g, 0)),
        ],
        out_specs=pl.BlockSpec((1, 1, _KG, _M), lambda b, g: (b, g, 0, 0)),
        out_shape=jax.ShapeDtypeStruct((_B, _GBK, _KG, _M), jnp.int32),
        scratch_shapes=[pltpu.VMEM((_KG, _NP), jnp.float32)],
    )(xf, center)


_DCH = _DIM // 16


def _sc_body(feat_hbm, xyzp_hbm, idx_hbm, ctrp_hbm, fts_out, nxyz_out,
             idx_v, rows_v, xrows_v, ctr_v, acc_v, sem1, sem2):
    wid = lax.axis_index("s") * _NC + lax.axis_index("c")
    base = wid * _GPW

    def group_body(gl, _):
        g = pl.multiple_of(base + gl, 1)
        o64 = pl.multiple_of(g * _M, 8)
        pltpu.sync_copy(idx_hbm.at[pl.ds(o64, _M)], idx_v)
        pltpu.async_copy(feat_hbm.at[idx_v], rows_v, sem1)
        pltpu.async_copy(xyzp_hbm.at[idx_v], xrows_v, sem2)
        pltpu.sync_copy(ctrp_hbm.at[g], ctr_v)
        pltpu.make_async_copy(feat_hbm.at[idx_v], rows_v, sem1).wait()

        def red_body(mi, accs):
            return tuple(
                accs[dc] + rows_v[mi, pl.ds(dc * 16, 16)]
                for dc in range(_DCH))

        accs = lax.fori_loop(
            0, _M, red_body,
            tuple(jnp.zeros((16,), jnp.float32) for _ in range(_DCH)))
        for dc in range(_DCH):
            acc_v[pl.ds(dc * 16, 16)] = accs[dc] * (1.0 / _M)
        pltpu.sync_copy(acc_v, fts_out.at[g])

        pltpu.make_async_copy(xyzp_hbm.at[idx_v], xrows_v, sem2).wait()
        cvec = ctr_v[...]

        def sub_body(mi, _c):
            xrows_v[mi, 0:16] = xrows_v[mi, 0:16] - cvec
            return 0

        lax.fori_loop(0, _M, sub_body, 0)
        pltpu.sync_copy(xrows_v, nxyz_out.at[pl.ds(o64, _M)])
        return 0

    lax.fori_loop(0, _GPW, group_body, 0)


def _sc_gather_mean(feat_tab, xyzp_tab, flat_idx, ctrp):
    mesh = plsc.VectorSubcoreMesh(core_axis_name="c", subcore_axis_name="s")
    return pl.kernel(
        _sc_body,
        mesh=mesh,
        out_type=[
            jax.ShapeDtypeStruct((_B * _G, _DIM), jnp.float32),
            jax.ShapeDtypeStruct((_B * _G * _M, 128), jnp.float32),
        ],
        scratch_types=[
            pltpu.VMEM((_M,), jnp.int32),
            pltpu.VMEM((_M, _DIM), jnp.float32),
            pltpu.VMEM((_M, 128), jnp.float32),
            pltpu.VMEM((16,), jnp.float32),
            pltpu.VMEM((_DIM,), jnp.float32),
            pltpu.SemaphoreType.DMA,
            pltpu.SemaphoreType.DMA,
        ],
    )(feat_tab, xyzp_tab, flat_idx, ctrp)


def kernel(xyzs, pointcloud_features, level):
    Bb, Nn, dim = pointcloud_features.shape
    xt = jnp.transpose(xyzs, (0, 2, 1))
    xr = xt.reshape(Bb, 3, _SUB, _LANES)
    center = _fps(xr)
    xtp = jnp.pad(xt, ((0, 0), (0, 0), (0, _NP - _N)),
                  constant_values=1e6)
    idx4 = _knn(xtp, center)
    idx = idx4.reshape(Bb, _G, _M)
    flat_idx = (idx + (jnp.arange(Bb, dtype=jnp.int32) * Nn)[:, None, None])
    flat_idx = flat_idx.reshape(-1).astype(jnp.int32)
    _USE_SC = True
    if _USE_SC:
        feat_tab = pointcloud_features.reshape(Bb * Nn, dim)
        xyzp_tab = jnp.pad(xyzs, ((0, 0), (0, 0), (0, 125))).reshape(Bb * Nn, 128)
        ctrp = jnp.pad(center, ((0, 0), (0, 0), (0, 13))).reshape(Bb * _G, 16)
        fts, nxyzp = _sc_gather_mean(feat_tab, xyzp_tab, flat_idx, ctrp)
        all_fts = fts.reshape(Bb, _G, dim)
        nxyz = nxyzp[:, :3].reshape(Bb, _G, _M, 3)
    else:
        bidx2 = jnp.arange(Bb)[:, None, None]
        all_fts = pointcloud_features[bidx2, idx].mean(-2)
        nxyz = xyzs[bidx2, idx] - center[:, :, None, :]
    all_fts_mask = jnp.ones((Bb, _G), dtype=pointcloud_features.dtype)
    return all_fts, all_fts_mask, center, nxyz


# speedup vs baseline: 7.4263x; 1.1360x over previous
import functools

import jax
import jax.numpy as jnp
from jax import lax
from jax.experimental import pallas as pl
from jax.experimental.pallas import tpu as pltpu
from jax.experimental.pallas import tpu_sc as plsc

_B = 2
_N = 40000
_DIM = 768
_G = 256
_M = 64

_SUB = 8
_LANES = _N // _SUB

_NC = 2
_NS = 16
_NW = _NC * _NS
_GPW = (_B * _G) // _NW


def _fps_body(xr_ref, ctr_ref):
    xyz_b = [(xr_ref[b, 0], xr_ref[b, 1], xr_ref[b, 2]) for b in range(_B)]
    r_iota = lax.broadcasted_iota(jnp.int32, (_SUB, _LANES), 0)
    c_iota = lax.broadcasted_iota(jnp.int32, (_SUB, _LANES), 1)
    flat = r_iota * _LANES + c_iota

    def body(i, carry):
        new = []
        for b in range(_B):
            x, y, z = xyz_b[b]
            dist_run, far = carry[b]
            mask = flat == far
            cx = jnp.sum(jnp.where(mask, x, 0.0))
            cy = jnp.sum(jnp.where(mask, y, 0.0))
            cz = jnp.sum(jnp.where(mask, z, 0.0))
            ctr_ref[b, pl.ds(i, 1), :] = jnp.concatenate(
                [cx.reshape(1, 1), cy.reshape(1, 1), cz.reshape(1, 1)], axis=1)
            d = (x - cx) ** 2 + (y - cy) ** 2 + (z - cz) ** 2
            dist_run = jnp.minimum(dist_run, d)
            m = jnp.max(dist_run)
            far2 = jnp.min(jnp.where(dist_run == m, flat, jnp.int32(2 ** 30)))
            new.append((dist_run, far2))
        return tuple(new)

    lax.fori_loop(0, _G, body, tuple(
        (jnp.full((_SUB, _LANES), 1e10, jnp.float32), jnp.int32(0))
        for _ in range(_B)))


def _fps(xr):
    return pl.pallas_call(
        _fps_body,
        grid=(1,),
        in_specs=[pl.BlockSpec((_B, 3, _SUB, _LANES), lambda b: (0, 0, 0, 0))],
        out_specs=pl.BlockSpec((_B, _G, 3), lambda b: (0, 0, 0)),
        out_shape=jax.ShapeDtypeStruct((_B, _G, 3), jnp.float32),
    )(xr)


_KG = 64
_GBK = _G // _KG
_NP = 40960


def _knn_body(xf_ref, ctr_ref, idx_ref, d_ref):
    x = xf_ref[0, 0:1, :]
    y = xf_ref[0, 1:2, :]
    z = xf_ref[0, 2:3, :]
    cx = ctr_ref[0, :, 0:1]
    cy = ctr_ref[0, :, 1:2]
    cz = ctr_ref[0, :, 2:3]
    xs = x * x + y * y + z * z
    cs = cx * cx + cy * cy + cz * cz
    xb = x.astype(jnp.bfloat16).astype(jnp.float32)
    yb = y.astype(jnp.bfloat16).astype(jnp.float32)
    zb = z.astype(jnp.bfloat16).astype(jnp.float32)
    cxb = cx.astype(jnp.bfloat16).astype(jnp.float32)
    cyb = cy.astype(jnp.bfloat16).astype(jnp.float32)
    czb = cz.astype(jnp.bfloat16).astype(jnp.float32)
    lane = lax.broadcasted_iota(jnp.int32, (_KG, _NP), 1)
    d2 = cs - 2.0 * (cxb * xb + cyb * yb + czb * zb) + xs
    d_ref[...] = jnp.where(lane < _N, d2, jnp.inf)
    col = lax.broadcasted_iota(jnp.int32, (_KG, _M), 1)

    def body(j, acc):
        d = d_ref[...]
        m = jnp.min(d, axis=1, keepdims=True)
        sel = jnp.where(d == m, lane, jnp.int32(2 ** 30))
        idx8 = jnp.min(sel, axis=1, keepdims=True)
        d_ref[...] = jnp.where(lane == idx8, jnp.inf, d)
        return jnp.where(col == j, idx8, acc)

    acc = lax.fori_loop(0, _M, body,
                        jnp.zeros((_KG, _M), jnp.int32))
    idx_ref[0, 0] = acc


def _knn(xf, center):
    return pl.pallas_call(
        _knn_body,
        grid=(_B, _GBK),
        in_specs=[
            pl.BlockSpec((1, 3, _NP), lambda b, g: (b, 0, 0)),
            pl.BlockSpec((1, _KG, 3), lambda b, g: (b, g, 0)),
        ],
        out_specs=pl.BlockSpec((1, 1, _KG, _M), lambda b, g: (b, g, 0, 0)),
        out_shape=jax.ShapeDtypeStruct((_B, _GBK, _KG, _M), jnp.int32),
        scratch_shapes=[pltpu.VMEM((_KG, _NP), jnp.float32)],
    )(xf, center)


_DCH = _DIM // 16


def _sc_body(feat_hbm, xyzp_hbm, idx_hbm, ctrp_hbm, fts_out, nxyz_out,
             idx_v, rows_v, xrows_v, ctr_v, acc_v, sem1, sem2):
    wid = lax.axis_index("s") * _NC + lax.axis_index("c")
    base = wid * _GPW

    def group_body(gl, _):
        g = pl.multiple_of(base + gl, 1)
        o64 = pl.multiple_of(g * _M, 8)
        pltpu.sync_copy(idx_hbm.at[pl.ds(o64, _M)], idx_v)
        pltpu.async_copy(feat_hbm.at[idx_v], rows_v, sem1)
        pltpu.async_copy(xyzp_hbm.at[idx_v], xrows_v, sem2)
        pltpu.sync_copy(ctrp_hbm.at[g], ctr_v)
        pltpu.make_async_copy(feat_hbm.at[idx_v], rows_v, sem1).wait()

        def red_body(mi, accs):
            return tuple(
                accs[dc] + rows_v[mi, pl.ds(dc * 16, 16)]
                for dc in range(_DCH))

        accs = lax.fori_loop(
            0, _M, red_body,
            tuple(jnp.zeros((16,), jnp.float32) for _ in range(_DCH)))
        for dc in range(_DCH):
            acc_v[pl.ds(dc * 16, 16)] = accs[dc] * (1.0 / _M)
        pltpu.sync_copy(acc_v, fts_out.at[g])

        pltpu.make_async_copy(xyzp_hbm.at[idx_v], xrows_v, sem2).wait()
        cvec = ctr_v[...]

        def sub_body(mi, _c):
            xrows_v[mi, 0:16] = xrows_v[mi, 0:16] - cvec
            return 0

        lax.fori_loop(0, _M, sub_body, 0)
        pltpu.sync_copy(xrows_v, nxyz_out.at[pl.ds(o64, _M)])
        return 0

    lax.fori_loop(0, _GPW, group_body, 0)


def _sc_gather_mean(feat_tab, xyzp_tab, flat_idx, ctrp):
    mesh = plsc.VectorSubcoreMesh(core_axis_name="c", subcore_axis_name="s")
    return pl.kernel(
        _sc_body,
        mesh=mesh,
        out_type=[
            jax.ShapeDtypeStruct((_B * _G, _DIM), jnp.float32),
            jax.ShapeDtypeStruct((_B * _G * _M, 128), jnp.float32),
        ],
        scratch_types=[
            pltpu.VMEM((_M,), jnp.int32),
            pltpu.VMEM((_M, _DIM), jnp.float32),
            pltpu.VMEM((_M, 128), jnp.float32),
            pltpu.VMEM((16,), jnp.float32),
            pltpu.VMEM((_DIM,), jnp.float32),
            pltpu.SemaphoreType.DMA,
            pltpu.SemaphoreType.DMA,
        ],
    )(feat_tab, xyzp_tab, flat_idx, ctrp)


def kernel(xyzs, pointcloud_features, level):
    Bb, Nn, dim = pointcloud_features.shape
    xt = jnp.transpose(xyzs, (0, 2, 1))
    xr = xt.reshape(Bb, 3, _SUB, _LANES)
    center = _fps(xr)
    xtp = jnp.pad(xt, ((0, 0), (0, 0), (0, _NP - _N)),
                  constant_values=1e6)
    idx4 = _knn(xtp, center)
    idx = idx4.reshape(Bb, _G, _M)
    flat_idx = (idx + (jnp.arange(Bb, dtype=jnp.int32) * Nn)[:, None, None])
    flat_idx = flat_idx.reshape(-1).astype(jnp.int32)
    _USE_SC = True
    if _USE_SC:
        feat_tab = pointcloud_features.reshape(Bb * Nn, dim)
        xyzp_tab = jnp.pad(xyzs, ((0, 0), (0, 0), (0, 125))).reshape(Bb * Nn, 128)
        ctrp = jnp.pad(center, ((0, 0), (0, 0), (0, 13))).reshape(Bb * _G, 16)
        fts, nxyzp = _sc_gather_mean(feat_tab, xyzp_tab, flat_idx, ctrp)
        all_fts = fts.reshape(Bb, _G, dim)
        nxyz = nxyzp[:, :3].reshape(Bb, _G, _M, 3)
    else:
        bidx2 = jnp.arange(Bb)[:, None, None]
        all_fts = pointcloud_features[bidx2, idx].mean(-2)
        nxyz = xyzs[bidx2, idx] - center[:, :, None, :]
    all_fts_mask = jnp.ones((Bb, _G), dtype=pointcloud_features.dtype)
    return all_fts, all_fts_mask, center, nxyz


# speedup vs baseline: 7.4290x; 1.0004x over previous
import functools

import jax
import jax.numpy as jnp
from jax import lax
from jax.experimental import pallas as pl
from jax.experimental.pallas import tpu as pltpu
from jax.experimental.pallas import tpu_sc as plsc

_B = 2
_N = 40000
_DIM = 768
_G = 256
_M = 64

_SUB = 8
_LANES = _N // _SUB

_NC = 2
_NS = 16
_NW = _NC * _NS
_GPW = (_B * _G) // _NW


def _fps_body(xr_ref, ctr_ref):
    xyz_b = [(xr_ref[b, 0], xr_ref[b, 1], xr_ref[b, 2]) for b in range(_B)]
    r_iota = lax.broadcasted_iota(jnp.int32, (_SUB, _LANES), 0)
    c_iota = lax.broadcasted_iota(jnp.int32, (_SUB, _LANES), 1)
    flat = r_iota * _LANES + c_iota

    def body(i, carry):
        new = []
        for b in range(_B):
            x, y, z = xyz_b[b]
            dist_run, far = carry[b]
            mask = flat == far
            cx = jnp.sum(jnp.where(mask, x, 0.0))
            cy = jnp.sum(jnp.where(mask, y, 0.0))
            cz = jnp.sum(jnp.where(mask, z, 0.0))
            ctr_ref[b, pl.ds(i, 1), :] = jnp.concatenate(
                [cx.reshape(1, 1), cy.reshape(1, 1), cz.reshape(1, 1)], axis=1)
            d = (x - cx) ** 2 + (y - cy) ** 2 + (z - cz) ** 2
            dist_run = jnp.minimum(dist_run, d)
            m = jnp.max(dist_run)
            far2 = jnp.min(jnp.where(dist_run == m, flat, jnp.int32(2 ** 30)))
            new.append((dist_run, far2))
        return tuple(new)

    lax.fori_loop(0, _G, body, tuple(
        (jnp.full((_SUB, _LANES), 1e10, jnp.float32), jnp.int32(0))
        for _ in range(_B)))


def _fps(xr):
    return pl.pallas_call(
        _fps_body,
        grid=(1,),
        in_specs=[pl.BlockSpec((_B, 3, _SUB, _LANES), lambda b: (0, 0, 0, 0))],
        out_specs=pl.BlockSpec((_B, _G, 3), lambda b: (0, 0, 0)),
        out_shape=jax.ShapeDtypeStruct((_B, _G, 3), jnp.float32),
    )(xr)


_KG = 64
_GBK = _G // _KG
_NP = 40960


def _knn_body(xf_ref, ctr_ref, idx_ref, d_ref):
    x = xf_ref[0, 0:1, :]
    y = xf_ref[0, 1:2, :]
    z = xf_ref[0, 2:3, :]
    cx = ctr_ref[0, :, 0:1]
    cy = ctr_ref[0, :, 1:2]
    cz = ctr_ref[0, :, 2:3]
    xs = x * x + y * y + z * z
    cs = cx * cx + cy * cy + cz * cz
    xb = x.astype(jnp.bfloat16).astype(jnp.float32)
    yb = y.astype(jnp.bfloat16).astype(jnp.float32)
    zb = z.astype(jnp.bfloat16).astype(jnp.float32)
    cxb = cx.astype(jnp.bfloat16).astype(jnp.float32)
    cyb = cy.astype(jnp.bfloat16).astype(jnp.float32)
    czb = cz.astype(jnp.bfloat16).astype(jnp.float32)
    lane = lax.broadcasted_iota(jnp.int32, (_KG, _NP), 1)
    d2 = cs - 2.0 * (cxb * xb + cyb * yb + czb * zb) + xs
    d_ref[...] = jnp.where(lane < _N, d2, jnp.inf)
    col = lax.broadcasted_iota(jnp.int32, (_KG, _M), 1)

    def body(j, acc):
        d = d_ref[...]
        m = jnp.min(d, axis=1, keepdims=True)
        sel = jnp.where(d == m, lane, jnp.int32(2 ** 30))
        idx8 = jnp.min(sel, axis=1, keepdims=True)
        d_ref[...] = jnp.where(lane == idx8, jnp.inf, d)
        return jnp.where(col == j, idx8, acc)

    acc = lax.fori_loop(0, _M, body,
                        jnp.zeros((_KG, _M), jnp.int32))
    idx_ref[0, 0] = acc


def _knn(xf, center):
    return pl.pallas_call(
        _knn_body,
        grid=(_B, _GBK),
        in_specs=[
            pl.BlockSpec((1, 3, _NP), lambda b, g: (b, 0, 0)),
            pl.BlockSpec((1, _KG, 3), lambda b, g: (b, g, 0)),
        ],
        out_specs=pl.BlockSpec((1, 1, _KG, _M), lambda b, g: (b, g, 0, 0)),
        out_shape=jax.ShapeDtypeStruct((_B, _GBK, _KG, _M), jnp.int32),
        scratch_shapes=[pltpu.VMEM((_KG, _NP), jnp.float32)],
    )(xf, center)


_DCH = _DIM // 16


def _sc_body(feat_hbm, xyzp_hbm, idx_hbm, ctrp_hbm, fts_out, nxyz_out,
             idx_v, rows_v, xrows_v, ctr_v, acc_v, sem1, sem2):
    wid = lax.axis_index("s") * _NC + lax.axis_index("c")
    base = wid * _GPW

    def group_body(gl, _):
        g = pl.multiple_of(base + gl, 1)
        o64 = pl.multiple_of(g * _M, 8)
        pltpu.sync_copy(idx_hbm.at[pl.ds(o64, _M)], idx_v)
        pltpu.async_copy(feat_hbm.at[idx_v], rows_v, sem1)
        pltpu.async_copy(xyzp_hbm.at[idx_v], xrows_v, sem2)
        pltpu.sync_copy(ctrp_hbm.at[g], ctr_v)
        pltpu.make_async_copy(feat_hbm.at[idx_v], rows_v, sem1).wait()

        def red_body(mi, accs):
            return tuple(
                accs[dc] + rows_v[mi, pl.ds(dc * 16, 16)]
                for dc in range(_DCH))

        accs = lax.fori_loop(
            0, _M, red_body,
            tuple(jnp.zeros((16,), jnp.float32) for _ in range(_DCH)))
        for dc in range(_DCH):
            acc_v[pl.ds(dc * 16, 16)] = accs[dc] * (1.0 / _M)
        pltpu.sync_copy(acc_v, fts_out.at[g])

        pltpu.make_async_copy(xyzp_hbm.at[idx_v], xrows_v, sem2).wait()
        cvec = ctr_v[...]

        def sub_body(mi, _c):
            xrows_v[mi, 0:16] = xrows_v[mi, 0:16] - cvec
            return 0

        lax.fori_loop(0, _M, sub_body, 0)
        pltpu.sync_copy(xrows_v, nxyz_out.at[pl.ds(o64, _M)])
        return 0

    lax.fori_loop(0, _GPW, group_body, 0)


def _sc_gather_mean(feat_tab, xyzp_tab, flat_idx, ctrp):
    mesh = plsc.VectorSubcoreMesh(core_axis_name="c", subcore_axis_name="s")
    return pl.kernel(
        _sc_body,
        mesh=mesh,
        out_type=[
            jax.ShapeDtypeStruct((_B * _G, _DIM), jnp.float32),
            jax.ShapeDtypeStruct((_B * _G * _M, 128), jnp.float32),
        ],
        scratch_types=[
            pltpu.VMEM((_M,), jnp.int32),
            pltpu.VMEM((_M, _DIM), jnp.float32),
            pltpu.VMEM((_M, 128), jnp.float32),
            pltpu.VMEM((16,), jnp.float32),
            pltpu.VMEM((_DIM,), jnp.float32),
            pltpu.SemaphoreType.DMA,
            pltpu.SemaphoreType.DMA,
        ],
    )(feat_tab, xyzp_tab, flat_idx, ctrp)


def kernel(xyzs, pointcloud_features, level):
    Bb, Nn, dim = pointcloud_features.shape
    xt = jnp.transpose(xyzs, (0, 2, 1))
    xr = xt.reshape(Bb, 3, _SUB, _LANES)
    center = _fps(xr)
    xtp = jnp.pad(xt, ((0, 0), (0, 0), (0, _NP - _N)),
                  constant_values=1e6)
    idx4 = _knn(xtp, center)
    idx = idx4.reshape(Bb, _G, _M)
    flat_idx = (idx + (jnp.arange(Bb, dtype=jnp.int32) * Nn)[:, None, None])
    flat_idx = flat_idx.reshape(-1).astype(jnp.int32)
    feat_tab = pointcloud_features.reshape(Bb * Nn, dim)
    xyzp_tab = jnp.pad(xyzs, ((0, 0), (0, 0), (0, 125))).reshape(Bb * Nn, 128)
    ctrp = jnp.pad(center, ((0, 0), (0, 0), (0, 13))).reshape(Bb * _G, 16)
    fts, nxyzp = _sc_gather_mean(feat_tab, xyzp_tab, flat_idx, ctrp)
    all_fts = fts.reshape(Bb, _G, dim)
    nxyz = nxyzp[:, :3].reshape(Bb, _G, _M, 3)
    all_fts_mask = jnp.ones((Bb, _G), dtype=pointcloud_features.dtype)
    return all_fts, all_fts_mask, center, nxyz
